# Initial kernel scaffold; baseline (speedup 1.0000x reference)
import jax
import jax.numpy as jnp
from jax.experimental import pallas as pl


def kernel(inputs, edges, edge_weights, W1, b1, W2, b2):
    raise NotImplementedError("write your pallas kernel here")


# speedup vs baseline: 5.0224x; 5.0224x over previous
import functools

import jax
import jax.numpy as jnp
from jax import lax
from jax.experimental import pallas as pl
from jax.experimental.pallas import tpu as pltpu
from jax.experimental.pallas import tpu_sc as plsc

_NS = 16
_NC = 2
_NW = _NC * _NS
_CHUNK = 128
_W = 144


def _tc_message_ffn(x_ref, w_ref, b_ref, o_ref):
    t = jnp.tanh(
        jnp.dot(x_ref[...], w_ref[...], preferred_element_type=jnp.float32)
        + b_ref[...]
    )
    ones = jnp.ones((t.shape[0], _W - t.shape[1]), jnp.float32)
    o_ref[...] = jnp.concatenate([t, ones], axis=1)


def _tc_update_ffn(x_ref, p_ref, w2a_ref, w2b_ref, b_ref, o_ref):
    ssum = p_ref[0] + p_ref[1]
    cnt = jnp.maximum(ssum[:, 128:129], 1.0)
    agg = ssum[:, :128] / cnt
    o_ref[...] = jnp.tanh(
        jnp.dot(x_ref[...], w2a_ref[...], preferred_element_type=jnp.float32)
        + jnp.dot(agg, w2b_ref[...], preferred_element_type=jnp.float32)
        + b_ref[...]
    )


def _make_sc_aggregate(np_, ch):
    stripe = np_ // _NS
    mesh = plsc.VectorSubcoreMesh(core_axis_name="c", subcore_axis_name="s")

    @functools.partial(
        pl.kernel,
        mesh=mesh,
        out_type=jax.ShapeDtypeStruct((_NC, np_, _W), jnp.float32),
        scratch_types=[
            pltpu.VMEM((ch, _CHUNK), jnp.int32),
            pltpu.VMEM((ch, _CHUNK), jnp.int32),
            pltpu.VMEM((_CHUNK, _W), jnp.float32),
            pltpu.VMEM_SHARED((np_, _W), jnp.float32),
            pltpu.SemaphoreType.DMA,
        ],
        compiler_params=pltpu.CompilerParams(use_tc_tiling_on_sc=False),
    )
    def sc_aggregate(yext, nbr, dst, zeros, out, nbr_v, dst_v, rows_v, acc, sem):
        c = lax.axis_index("c")
        s = lax.axis_index("s")
        w = c * _NS + s
        row0 = s * stripe
        pltpu.sync_copy(zeros.at[pl.ds(row0, stripe)], acc.at[pl.ds(row0, stripe)])
        pltpu.sync_copy(nbr.at[w], nbr_v)
        pltpu.sync_copy(dst.at[w], dst_v)
        plsc.subcore_barrier()

        def body(j, carry):
            pltpu.async_copy(yext.at[nbr_v.at[j]], rows_v, sem).wait()
            pltpu.sync_copy(rows_v, acc.at[dst_v.at[j]], add=True)
            return carry

        lax.fori_loop(0, ch, body, 0)
        plsc.subcore_barrier()
        pltpu.sync_copy(
            acc.at[pl.ds(row0, stripe)], out.at[c, pl.ds(row0, stripe)]
        )

    return sc_aggregate


def kernel(inputs, edges, edge_weights, W1, b1, W2, b2):
    del edge_weights
    _, n, d = inputs.shape
    h = W1.shape[1]
    e = edges.shape[1]

    np_ = ((n + _NS * 8 - 1) // (_NS * 8)) * (_NS * 8)
    ch = -(-e // (_NW * _CHUNK))
    ep = _NW * ch * _CHUNK

    x = inputs[0]
    xp = jnp.pad(x, ((0, np_ - n), (0, 0)))
    nbr = jnp.pad(edges[1], (0, ep - e)).reshape(_NW, ch, _CHUNK)
    dst = jnp.pad(edges[0], (0, ep - e), constant_values=n).reshape(_NW, ch, _CHUNK)
    zeros = jnp.zeros((np_, _W), jnp.float32)

    yext = pl.pallas_call(
        _tc_message_ffn,
        out_shape=jax.ShapeDtypeStruct((np_, _W), jnp.float32),
    )(xp, W1, b1.reshape(1, h))

    partials = _make_sc_aggregate(np_, ch)(yext, nbr, dst, zeros)

    out = pl.pallas_call(
        _tc_update_ffn,
        out_shape=jax.ShapeDtypeStruct((np_, h), jnp.float32),
    )(xp, partials, W2[:d], W2[d:], b2.reshape(1, h))

    return out[:n][None]


# speedup vs baseline: 5.7694x; 1.1487x over previous
import functools

import jax
import jax.numpy as jnp
from jax import lax
from jax.experimental import pallas as pl
from jax.experimental.pallas import tpu as pltpu
from jax.experimental.pallas import tpu_sc as plsc

_NS = 16
_NC = 2
_NW = _NC * _NS
_CHUNK = 64
_W = 144


def _tc_message_ffn(x_ref, w_ref, b_ref, o_ref):
    t = jnp.tanh(
        jnp.dot(x_ref[...], w_ref[...], preferred_element_type=jnp.float32)
        + b_ref[...]
    )
    ones = jnp.ones((t.shape[0], _W - t.shape[1]), jnp.float32)
    o_ref[...] = jnp.concatenate([t, ones], axis=1)


def _tc_update_ffn(x_ref, p_ref, w2a_ref, w2b_ref, b_ref, o_ref):
    ssum = p_ref[0] + p_ref[1]
    cnt = jnp.maximum(ssum[:, 128:129], 1.0)
    agg = ssum[:, :128] / cnt
    o_ref[...] = jnp.tanh(
        jnp.dot(x_ref[...], w2a_ref[...], preferred_element_type=jnp.float32)
        + jnp.dot(agg, w2b_ref[...], preferred_element_type=jnp.float32)
        + b_ref[...]
    )


def _make_sc_aggregate(np_, ch):
    stripe = np_ // _NS
    mesh = plsc.VectorSubcoreMesh(core_axis_name="c", subcore_axis_name="s")

    @functools.partial(
        pl.kernel,
        mesh=mesh,
        out_type=jax.ShapeDtypeStruct((_NC, np_, _W), jnp.float32),
        scratch_types=[
            pltpu.VMEM((ch, _CHUNK), jnp.int32),
            pltpu.VMEM((ch, _CHUNK), jnp.int32),
            pltpu.VMEM((_CHUNK, _W), jnp.float32),
            pltpu.VMEM((_CHUNK, _W), jnp.float32),
            pltpu.VMEM_SHARED((np_, _W), jnp.float32),
            pltpu.SemaphoreType.DMA,
            pltpu.SemaphoreType.DMA,
        ],
        compiler_params=pltpu.CompilerParams(use_tc_tiling_on_sc=False),
    )
    def sc_aggregate(
        yext, nbr, dst, zeros, out, nbr_v, dst_v, rows0_v, rows1_v, acc, sem0, sem1
    ):
        c = lax.axis_index("c")
        s = lax.axis_index("s")
        w = c * _NS + s
        row0 = s * stripe
        pltpu.sync_copy(zeros.at[pl.ds(row0, stripe)], acc.at[pl.ds(row0, stripe)])
        pltpu.sync_copy(nbr.at[w], nbr_v)
        pltpu.sync_copy(dst.at[w], dst_v)
        plsc.subcore_barrier()

        pltpu.async_copy(yext.at[nbr_v.at[0]], rows0_v, sem0)

        def body(g, carry):
            j0 = 2 * g
            pltpu.async_copy(yext.at[nbr_v.at[j0 + 1]], rows1_v, sem1)
            pltpu.make_async_copy(yext.at[nbr_v.at[j0]], rows0_v, sem0).wait()
            pltpu.sync_copy(rows0_v, acc.at[dst_v.at[j0]], add=True)

            @pl.when(j0 + 2 < ch)
            def _():
                pltpu.async_copy(yext.at[nbr_v.at[j0 + 2]], rows0_v, sem0)

            pltpu.make_async_copy(yext.at[nbr_v.at[j0 + 1]], rows1_v, sem1).wait()
            pltpu.sync_copy(rows1_v, acc.at[dst_v.at[j0 + 1]], add=True)
            return carry

        lax.fori_loop(0, ch // 2, body, 0)
        plsc.subcore_barrier()
        pltpu.sync_copy(
            acc.at[pl.ds(row0, stripe)], out.at[c, pl.ds(row0, stripe)]
        )

    return sc_aggregate


def kernel(inputs, edges, edge_weights, W1, b1, W2, b2):
    del edge_weights
    _, n, d = inputs.shape
    h = W1.shape[1]
    e = edges.shape[1]

    np_ = ((n + _NS * 8 - 1) // (_NS * 8)) * (_NS * 8)
    ch = -(-e // (_NW * _CHUNK))
    ch += ch % 2
    ep = _NW * ch * _CHUNK

    x = inputs[0]
    xp = jnp.pad(x, ((0, np_ - n), (0, 0)))
    nbr = jnp.pad(edges[1], (0, ep - e)).reshape(_NW, ch, _CHUNK)
    dst = jnp.pad(edges[0], (0, ep - e), constant_values=n).reshape(_NW, ch, _CHUNK)
    zeros = jnp.zeros((np_, _W), jnp.float32)

    yext = pl.pallas_call(
        _tc_message_ffn,
        out_shape=jax.ShapeDtypeStruct((np_, _W), jnp.float32),
    )(xp, W1, b1.reshape(1, h))

    partials = _make_sc_aggregate(np_, ch)(yext, nbr, dst, zeros)

    out = pl.pallas_call(
        _tc_update_ffn,
        out_shape=jax.ShapeDtypeStruct((np_, h), jnp.float32),
    )(xp, partials, W2[:d], W2[d:], b2.reshape(1, h))

    return out[:n][None]
